# Initial kernel scaffold; baseline (speedup 1.0000x reference)
import jax
import jax.numpy as jnp
from jax.experimental import pallas as pl


def kernel(x, edge_index, W, b):
    raise NotImplementedError("write your pallas kernel here")


# speedup vs baseline: 4.0553x; 4.0553x over previous
import functools

import jax
import jax.numpy as jnp
from jax import lax
from jax.experimental import pallas as pl
from jax.experimental.pallas import tpu as pltpu
from jax.experimental.pallas import tpu_sc as plsc

N_NODES = 10000
N_EDGES = 160000
D_FEAT = 256
NUM_HEADS = 8
CH = 128


def _tc_tables(x, W2, b2):
    M = x.shape[0]
    BM = 1000
    H2 = 2 * NUM_HEADS

    def body(x_ref, w_ref, b_ref, t1_ref, t2_ref):
        y = (
            jnp.dot(x_ref[...], w_ref[...], preferred_element_type=jnp.float32)
            + b_ref[...]
        )
        t1_ref[...] = y
        t2_ref[...] = jnp.concatenate(
            [y[:, NUM_HEADS:], y[:, :NUM_HEADS]], axis=1
        )

    return pl.pallas_call(
        body,
        grid=(M // BM,),
        in_specs=[
            pl.BlockSpec((BM, D_FEAT), lambda i: (i, 0)),
            pl.BlockSpec((D_FEAT, H2), lambda i: (0, 0)),
            pl.BlockSpec((1, H2), lambda i: (0, 0)),
        ],
        out_specs=[
            pl.BlockSpec((BM, H2), lambda i: (i, 0)),
            pl.BlockSpec((BM, H2), lambda i: (i, 0)),
        ],
        out_shape=[
            jax.ShapeDtypeStruct((M, H2), jnp.float32),
            jax.ShapeDtypeStruct((M, H2), jnp.float32),
        ],
    )(x, W2, b2)


def _sc_edge_combine(T1, T2, o2, d2):
    info = plsc.get_sparse_core_info()
    NW = info.num_cores * info.num_subcores
    R = o2.shape[0]
    RPW = R // NW
    mesh = plsc.VectorSubcoreMesh(core_axis_name="c", subcore_axis_name="s")

    @functools.partial(
        pl.kernel,
        out_type=jax.ShapeDtypeStruct((R * CH * 16,), jnp.float32),
        mesh=mesh,
        compiler_params=pltpu.CompilerParams(use_tc_tiling_on_sc=False),
        scratch_types=[
            pltpu.VMEM((RPW, CH), jnp.int32),
            pltpu.VMEM((RPW, CH), jnp.int32),
            pltpu.VMEM((CH, 16), jnp.float32),
            pltpu.VMEM((CH, 16), jnp.float32),
            pltpu.VMEM((CH, 16), jnp.float32),
            pltpu.VMEM((CH, 16), jnp.float32),
            pltpu.VMEM((CH * 16,), jnp.float32),
            pltpu.VMEM((CH * 16,), jnp.float32),
            pltpu.SemaphoreType.DMA,
            pltpu.SemaphoreType.DMA,
            pltpu.SemaphoreType.DMA,
            pltpu.SemaphoreType.DMA,
            pltpu.SemaphoreType.DMA,
            pltpu.SemaphoreType.DMA,
        ],
    )
    def k(t1_hbm, t2_hbm, o_hbm, d_hbm, out_hbm, io, idd, a0, a1, b0, b1,
          o0, o1, sa0, sa1, sb0, sb1, so0, so1):
        wid = lax.axis_index("s") * info.num_cores + lax.axis_index("c")
        base_r = wid * RPW
        pltpu.sync_copy(o_hbm.at[pl.ds(base_r, RPW)], io)
        pltpu.sync_copy(d_hbm.at[pl.ds(base_r, RPW)], idd)

        A = [a0, a1]
        B = [b0, b1]
        O = [o0, o1]
        SA = [sa0, sa1]
        SB = [sb0, sb1]
        SO = [so0, so1]

        def issue(j):
            p = j % 2
            ca = pltpu.make_async_copy(t1_hbm.at[io.at[j]], A[p], SA[p])
            ca.start()
            cb = pltpu.make_async_copy(t2_hbm.at[idd.at[j]], B[p], SB[p])
            cb.start()
            return ca, cb

        def compute(p):
            def body(i, _):
                O[p][pl.ds(i * 16, 16)] = A[p][i, :] + B[p][i, :]
                return 0

            lax.fori_loop(0, CH, body, 0)

        copies = [None] * RPW
        outc = [None] * RPW
        copies[0] = issue(0)
        for j in range(RPW):
            p = j % 2
            if j + 1 < RPW:
                copies[j + 1] = issue(j + 1)
            ca, cb = copies[j]
            ca.wait()
            cb.wait()
            if j >= 2:
                outc[j - 2].wait()
            compute(p)
            oc = pltpu.make_async_copy(
                O[p],
                out_hbm.at[pl.ds((base_r + j) * CH * 16, CH * 16)],
                SO[p],
            )
            oc.start()
            outc[j] = oc
        outc[RPW - 2].wait()
        outc[RPW - 1].wait()

    return k(T1, T2, o2, d2)


def kernel(x, edge_index, W, b):
    W2 = jnp.concatenate([W[:D_FEAT], W[D_FEAT:]], axis=1)
    b2 = jnp.concatenate([b, jnp.zeros((NUM_HEADS,), jnp.float32)])
    T1, T2 = _tc_tables(x, W2, b2.reshape(1, 2 * NUM_HEADS))

    ep = ((N_EDGES + 32 * CH - 1) // (32 * CH)) * (32 * CH)
    pad = ep - N_EDGES
    o = edge_index[0].astype(jnp.int32)
    d = edge_index[1].astype(jnp.int32)
    if pad:
        z = jnp.zeros((pad,), jnp.int32)
        o = jnp.concatenate([o, z])
        d = jnp.concatenate([d, z])
    o2 = o.reshape(ep // CH, CH)
    d2 = d.reshape(ep // CH, CH)

    out_flat = _sc_edge_combine(T1, T2, o2, d2)
    return out_flat.reshape(ep, 16)[:N_EDGES, :NUM_HEADS]


# speedup vs baseline: 5.1985x; 1.2819x over previous
import functools

import jax
import jax.numpy as jnp
from jax import lax
from jax.experimental import pallas as pl
from jax.experimental.pallas import tpu as pltpu
from jax.experimental.pallas import tpu_sc as plsc

N_NODES = 10000
N_EDGES = 160000
D_FEAT = 256
NUM_HEADS = 8
CH = 125


def _tc_tables(x, W2, b2):
    M = x.shape[0]
    BM = 1000
    H2 = 2 * NUM_HEADS

    def body(x_ref, w_ref, b_ref, t1_ref, t2_ref):
        y = (
            jnp.dot(x_ref[...], w_ref[...], preferred_element_type=jnp.float32)
            + b_ref[...]
        )
        t1_ref[...] = y
        t2_ref[...] = jnp.concatenate(
            [y[:, NUM_HEADS:], y[:, :NUM_HEADS]], axis=1
        )

    return pl.pallas_call(
        body,
        grid=(M // BM,),
        in_specs=[
            pl.BlockSpec((BM, D_FEAT), lambda i: (i, 0)),
            pl.BlockSpec((D_FEAT, H2), lambda i: (0, 0)),
            pl.BlockSpec((1, H2), lambda i: (0, 0)),
        ],
        out_specs=[
            pl.BlockSpec((BM, H2), lambda i: (i, 0)),
            pl.BlockSpec((BM, H2), lambda i: (i, 0)),
        ],
        out_shape=[
            jax.ShapeDtypeStruct((M, H2), jnp.float32),
            jax.ShapeDtypeStruct((M, H2), jnp.float32),
        ],
    )(x, W2, b2)


def _sc_edge_combine(T1, T2, o2, d2):
    info = plsc.get_sparse_core_info()
    NW = info.num_cores * info.num_subcores
    R = o2.shape[0]
    RPW = R // NW
    mesh = plsc.VectorSubcoreMesh(core_axis_name="c", subcore_axis_name="s")

    @functools.partial(
        pl.kernel,
        out_type=jax.ShapeDtypeStruct((R * CH * NUM_HEADS,), jnp.float32),
        mesh=mesh,
        compiler_params=pltpu.CompilerParams(use_tc_tiling_on_sc=False),
        scratch_types=[
            pltpu.VMEM((RPW, CH), jnp.int32),
            pltpu.VMEM((RPW, CH), jnp.int32),
            pltpu.VMEM((CH, 16), jnp.float32),
            pltpu.VMEM((CH, 16), jnp.float32),
            pltpu.VMEM((CH, 16), jnp.float32),
            pltpu.VMEM((CH, 16), jnp.float32),
            pltpu.VMEM((CH * NUM_HEADS + 8,), jnp.float32),
            pltpu.VMEM((CH * NUM_HEADS + 8,), jnp.float32),
            pltpu.SemaphoreType.DMA,
            pltpu.SemaphoreType.DMA,
            pltpu.SemaphoreType.DMA,
            pltpu.SemaphoreType.DMA,
            pltpu.SemaphoreType.DMA,
            pltpu.SemaphoreType.DMA,
        ],
    )
    def k(t1_hbm, t2_hbm, o_hbm, d_hbm, out_hbm, io, idd, a0, a1, b0, b1,
          o0, o1, sa0, sa1, sb0, sb1, so0, so1):
        wid = lax.axis_index("s") * info.num_cores + lax.axis_index("c")
        base_r = wid * RPW
        pltpu.sync_copy(o_hbm.at[pl.ds(base_r, RPW)], io)
        pltpu.sync_copy(d_hbm.at[pl.ds(base_r, RPW)], idd)

        A = [a0, a1]
        B = [b0, b1]
        O = [o0, o1]
        SA = [sa0, sa1]
        SB = [sb0, sb1]
        SO = [so0, so1]

        def issue(j):
            p = j % 2
            ca = pltpu.make_async_copy(t1_hbm.at[io.at[j]], A[p], SA[p])
            ca.start()
            cb = pltpu.make_async_copy(t2_hbm.at[idd.at[j]], B[p], SB[p])
            cb.start()
            return ca, cb

        def compute(p):
            def body(i, _):
                O[p][pl.ds(i * NUM_HEADS, 16)] = A[p][i, :] + B[p][i, :]
                return 0

            lax.fori_loop(0, CH, body, 0)

        copies = [None] * RPW
        outc = [None] * RPW
        copies[0] = issue(0)
        for j in range(RPW):
            p = j % 2
            if j + 1 < RPW:
                copies[j + 1] = issue(j + 1)
            ca, cb = copies[j]
            ca.wait()
            cb.wait()
            if j >= 2:
                outc[j - 2].wait()
            compute(p)
            oc = pltpu.make_async_copy(
                O[p].at[pl.ds(0, CH * NUM_HEADS)],
                out_hbm.at[pl.ds((base_r + j) * CH * NUM_HEADS, CH * NUM_HEADS)],
                SO[p],
            )
            oc.start()
            outc[j] = oc
        outc[RPW - 2].wait()
        outc[RPW - 1].wait()

    return k(T1, T2, o2, d2)


def kernel(x, edge_index, W, b):
    W2 = jnp.concatenate([W[:D_FEAT], W[D_FEAT:]], axis=1)
    b2 = jnp.concatenate([b, jnp.zeros((NUM_HEADS,), jnp.float32)])
    T1, T2 = _tc_tables(x, W2, b2.reshape(1, 2 * NUM_HEADS))

    ei = edge_index.astype(jnp.int32).reshape(2, N_EDGES // CH, CH)
    out_flat = _sc_edge_combine(T1, T2, ei[0], ei[1])
    return out_flat.reshape(N_EDGES, NUM_HEADS)
